# baseline (speedup 1.0000x reference)
import functools

import jax
import jax.numpy as jnp
from jax import lax
from jax.experimental import pallas as pl
from jax.experimental.pallas import tpu as pltpu
from jax.experimental.pallas import tpu_sc as plsc

N_BINS = 15


def _tc_body(logits_ref, labels_ref, conf_ref, acc_ref):
    x = logits_ref[...]
    r, c = x.shape
    m = jnp.max(x, axis=1, keepdims=True)
    e = jnp.exp(x - m)
    s = jnp.sum(e, axis=1)
    conf = 1.0 / s
    iota = lax.broadcasted_iota(jnp.int32, (r, c), 1)
    cand = jnp.where(x == m, iota, c)
    pred = jnp.min(cand, axis=1)
    lab = labels_ref[0, 0, :]
    acc = (pred == lab).astype(jnp.float32)
    conf_ref[0, 0, :] = conf
    acc_ref[0, 0, :] = acc


def _tc_stats(logits, labels32, block_rows):
    n, c = logits.shape
    g = n // block_rows
    labels3 = labels32.reshape(g, 1, block_rows)
    conf, acc = pl.pallas_call(
        _tc_body,
        grid=(g,),
        in_specs=[
            pl.BlockSpec((block_rows, c), lambda i: (i, 0)),
            pl.BlockSpec((1, 1, block_rows), lambda i: (i, 0, 0)),
        ],
        out_specs=[
            pl.BlockSpec((1, 1, block_rows), lambda i: (i, 0, 0)),
            pl.BlockSpec((1, 1, block_rows), lambda i: (i, 0, 0)),
        ],
        out_shape=[
            jax.ShapeDtypeStruct((g, 1, block_rows), jnp.float32),
            jax.ShapeDtypeStruct((g, 1, block_rows), jnp.float32),
        ],
    )(logits, labels3)
    return conf.reshape(n), acc.reshape(n)


def _sc_hist(conf, acc):
    n = conf.shape[0]
    info = plsc.get_sparse_core_info()
    nc, ns = info.num_cores, info.num_subcores
    nw = nc * ns
    chunk = n // nw

    mesh = plsc.VectorSubcoreMesh(core_axis_name="c", subcore_axis_name="s")

    @functools.partial(
        pl.kernel,
        mesh=mesh,
        out_type=jax.ShapeDtypeStruct((nw * 3 * 16,), jnp.float32),
        compiler_params=pltpu.CompilerParams(needs_layout_passes=False),
        scratch_types=[
            pltpu.VMEM((chunk,), jnp.float32),
            pltpu.VMEM((chunk,), jnp.float32),
            pltpu.VMEM((16,), jnp.float32),
            pltpu.VMEM((16,), jnp.float32),
            pltpu.VMEM((16,), jnp.float32),
        ],
    )
    def hist(conf_hbm, acc_hbm, out_hbm, conf_v, acc_v, cnt_v, sconf_v, sacc_v):
        wid = lax.axis_index("s") * nc + lax.axis_index("c")
        base = wid * chunk
        pltpu.sync_copy(conf_hbm.at[pl.ds(base, chunk)], conf_v)
        pltpu.sync_copy(acc_hbm.at[pl.ds(base, chunk)], acc_v)
        zeros = jnp.zeros((16,), jnp.float32)
        cnt_v[...] = zeros
        sconf_v[...] = zeros
        sacc_v[...] = zeros
        ones = jnp.ones((16,), jnp.float32)

        def body(i, carry):
            c16 = conf_v[pl.ds(i * 16, 16)]
            a16 = acc_v[pl.ds(i * 16, 16)]
            b = jnp.minimum((c16 * float(N_BINS)).astype(jnp.int32), N_BINS - 1)
            plsc.addupdate_scatter(cnt_v, [b], ones)
            plsc.addupdate_scatter(sconf_v, [b], c16)
            plsc.addupdate_scatter(sacc_v, [b], a16)
            return carry

        lax.fori_loop(0, chunk // 16, body, 0)
        obase = wid * 48
        pltpu.sync_copy(cnt_v, out_hbm.at[pl.ds(obase, 16)])
        pltpu.sync_copy(sconf_v, out_hbm.at[pl.ds(obase + 16, 16)])
        pltpu.sync_copy(sacc_v, out_hbm.at[pl.ds(obase + 32, 16)])

    return hist(conf, acc).reshape(nw, 3, 16)


def kernel(logits, labels):
    n, c = logits.shape
    labels32 = labels.astype(jnp.int32)
    conf, acc = _tc_stats(logits, labels32, block_rows=4096)
    parts = _sc_hist(conf, acc)
    stats = parts.sum(axis=0)
    cnt = stats[0, :N_BINS]
    sconf = stats[1, :N_BINS]
    sacc = stats[2, :N_BINS]
    safe = jnp.maximum(cnt, 1.0)
    term = jnp.abs(sconf / safe - sacc / safe) * (cnt / n)
    ece = jnp.sum(jnp.where(cnt > 0, term, 0.0))
    return ece.reshape(1)


# speedup vs baseline: 2.6853x; 2.6853x over previous
import functools

import jax
import jax.numpy as jnp
from jax import lax
from jax.experimental import pallas as pl
from jax.experimental.pallas import tpu as pltpu
from jax.experimental.pallas import tpu_sc as plsc

N_BINS = 15


def _tc_body(logits_ref, labels_ref, conf_ref, acc_ref):
    x = logits_ref[...]
    r, c = x.shape
    i0 = lax.broadcasted_iota(jnp.int32, (c, c), 0)
    i1 = lax.broadcasted_iota(jnp.int32, (c, c), 1)
    eye = (i0 == i1).astype(jnp.float32)
    xt = lax.dot_general(eye, x, (((1,), (1,)), ((), ())),
                         preferred_element_type=jnp.float32)
    m = jnp.max(xt, axis=0, keepdims=True)
    e = jnp.exp(xt - m)
    ones = jnp.ones((1, c), jnp.float32)
    s = lax.dot_general(ones, e, (((1,), (0,)), ((), ())),
                        preferred_element_type=jnp.float32)
    conf = 1.0 / s
    mask = (xt == m).astype(jnp.float32)
    ci = lax.broadcasted_iota(jnp.int32, (1, c), 1)
    w = lax.bitcast_convert_type((127 + 63 - ci) << 23, jnp.float32)
    pv = lax.dot_general(w, mask, (((1,), (0,)), ((), ())),
                         preferred_element_type=jnp.float32)
    ex = (lax.bitcast_convert_type(pv, jnp.int32) >> 23) - 127
    pred = 63 - ex
    lab = labels_ref[0]
    acc = (pred == lab).astype(jnp.float32)
    conf_ref[0] = conf
    acc_ref[0] = acc


def _tc_stats(logits, labels32, block_rows):
    n, c = logits.shape
    g = n // block_rows
    labels3 = labels32.reshape(g, 1, block_rows)
    conf, acc = pl.pallas_call(
        _tc_body,
        grid=(g,),
        in_specs=[
            pl.BlockSpec((block_rows, c), lambda i: (i, 0)),
            pl.BlockSpec((1, 1, block_rows), lambda i: (i, 0, 0)),
        ],
        out_specs=[
            pl.BlockSpec((1, 1, block_rows), lambda i: (i, 0, 0)),
            pl.BlockSpec((1, 1, block_rows), lambda i: (i, 0, 0)),
        ],
        out_shape=[
            jax.ShapeDtypeStruct((g, 1, block_rows), jnp.float32),
            jax.ShapeDtypeStruct((g, 1, block_rows), jnp.float32),
        ],
        compiler_params=pltpu.CompilerParams(
            dimension_semantics=("arbitrary",)),
    )(logits, labels3)
    return conf.reshape(n), acc.reshape(n)


def _sc_hist(conf, acc):
    n = conf.shape[0]
    info = plsc.get_sparse_core_info()
    nc, ns = info.num_cores, info.num_subcores
    nw = nc * ns
    chunk = n // nw

    mesh = plsc.VectorSubcoreMesh(core_axis_name="c", subcore_axis_name="s")

    @functools.partial(
        pl.kernel,
        mesh=mesh,
        out_type=jax.ShapeDtypeStruct((nw * 768,), jnp.float32),
        compiler_params=pltpu.CompilerParams(needs_layout_passes=False),
        scratch_types=[
            pltpu.VMEM((chunk,), jnp.float32),
            pltpu.VMEM((chunk,), jnp.float32),
            pltpu.VMEM((768,), jnp.float32),
        ],
    )
    def hist(conf_hbm, acc_hbm, out_hbm, conf_v, acc_v, hist_v):
        wid = lax.axis_index("s") * nc + lax.axis_index("c")
        base = wid * chunk
        pltpu.sync_copy(conf_hbm.at[pl.ds(base, chunk)], conf_v)
        pltpu.sync_copy(acc_hbm.at[pl.ds(base, chunk)], acc_v)
        zeros = jnp.zeros((16,), jnp.float32)
        for j in range(48):
            hist_v[pl.ds(j * 16, 16)] = zeros
        ones = jnp.ones((16,), jnp.float32)
        lane = lax.iota(jnp.int32, 16)

        def step(off):
            c16 = conf_v[pl.ds(off, 16)]
            a16 = acc_v[pl.ds(off, 16)]
            b = jnp.minimum((c16 * float(N_BINS)).astype(jnp.int32), N_BINS - 1)
            idx = b * 16 + lane
            plsc.addupdate_scatter(hist_v, [idx], ones)
            plsc.addupdate_scatter(hist_v, [idx + 256], c16)
            plsc.addupdate_scatter(hist_v, [idx + 512], a16)

        unroll = 4
        def body(i, carry):
            for u in range(unroll):
                step(i * (16 * unroll) + u * 16)
            return carry

        lax.fori_loop(0, chunk // (16 * unroll), body, 0)
        pltpu.sync_copy(hist_v, out_hbm.at[pl.ds(wid * 768, 768)])

    return hist(conf, acc).reshape(nw, 3, 16, 16)


def kernel(logits, labels):
    n, c = logits.shape
    labels32 = labels.astype(jnp.int32)
    conf, acc = _tc_stats(logits, labels32, block_rows=4096)
    parts = _sc_hist(conf, acc)
    stats = parts.sum(axis=(0, 3))
    cnt = stats[0, :N_BINS]
    sconf = stats[1, :N_BINS]
    sacc = stats[2, :N_BINS]
    safe = jnp.maximum(cnt, 1.0)
    term = jnp.abs(sconf / safe - sacc / safe) * (cnt / n)
    ece = jnp.sum(jnp.where(cnt > 0, term, 0.0))
    return ece.reshape(1)


# speedup vs baseline: 9.5670x; 3.5627x over previous
import functools

import jax
import jax.numpy as jnp
from jax import lax
from jax.experimental import pallas as pl
from jax.experimental.pallas import tpu as pltpu
from jax.experimental.pallas import tpu_sc as plsc

N_BINS = 15


def _tc_body(lt_ref, labels_ref, conf_ref, acc_ref):
    xt = lt_ref[...]
    c, r = xt.shape
    m = jnp.max(xt, axis=0, keepdims=True)
    e = jnp.exp(xt - m)
    ones = jnp.ones((1, c), jnp.float32)
    s = lax.dot_general(ones, e, (((1,), (0,)), ((), ())),
                        preferred_element_type=jnp.float32)
    conf = 1.0 / s
    mask = (xt == m).astype(jnp.float32)
    ci = lax.broadcasted_iota(jnp.int32, (1, c), 1)
    w = lax.bitcast_convert_type((127 + 63 - ci) << 23, jnp.float32)
    pv = lax.dot_general(w, mask, (((1,), (0,)), ((), ())),
                         preferred_element_type=jnp.float32)
    ex = (lax.bitcast_convert_type(pv, jnp.int32) >> 23) - 127
    pred = 63 - ex
    lab = labels_ref[0]
    acc = (pred == lab).astype(jnp.float32)
    conf_ref[0] = conf
    acc_ref[0] = acc


def _tc_stats(logits, labels32, block_rows):
    n, c = logits.shape
    g = n // block_rows
    lt = logits.T
    labels3 = labels32.reshape(g, 1, block_rows)
    conf, acc = pl.pallas_call(
        _tc_body,
        grid=(g,),
        in_specs=[
            pl.BlockSpec((c, block_rows), lambda i: (0, i)),
            pl.BlockSpec((1, 1, block_rows), lambda i: (i, 0, 0)),
        ],
        out_specs=[
            pl.BlockSpec((1, 1, block_rows), lambda i: (i, 0, 0)),
            pl.BlockSpec((1, 1, block_rows), lambda i: (i, 0, 0)),
        ],
        out_shape=[
            jax.ShapeDtypeStruct((g, 1, block_rows), jnp.float32),
            jax.ShapeDtypeStruct((g, 1, block_rows), jnp.float32),
        ],
        compiler_params=pltpu.CompilerParams(
            dimension_semantics=("arbitrary",)),
    )(lt, labels3)
    return conf.reshape(n), acc.reshape(n)


def _sc_hist(conf, acc):
    n = conf.shape[0]
    info = plsc.get_sparse_core_info()
    nc, ns = info.num_cores, info.num_subcores
    nw = nc * ns
    chunk = n // nw

    mesh = plsc.VectorSubcoreMesh(core_axis_name="c", subcore_axis_name="s")

    @functools.partial(
        pl.kernel,
        mesh=mesh,
        out_type=jax.ShapeDtypeStruct((nw * 768,), jnp.float32),
        compiler_params=pltpu.CompilerParams(needs_layout_passes=False),
        scratch_types=[
            pltpu.VMEM((chunk,), jnp.float32),
            pltpu.VMEM((chunk,), jnp.float32),
            pltpu.VMEM((768,), jnp.float32),
        ],
    )
    def hist(conf_hbm, acc_hbm, out_hbm, conf_v, acc_v, hist_v):
        wid = lax.axis_index("s") * nc + lax.axis_index("c")
        base = wid * chunk
        pltpu.sync_copy(conf_hbm.at[pl.ds(base, chunk)], conf_v)
        pltpu.sync_copy(acc_hbm.at[pl.ds(base, chunk)], acc_v)
        zeros = jnp.zeros((16,), jnp.float32)
        for j in range(48):
            hist_v[pl.ds(j * 16, 16)] = zeros
        ones = jnp.ones((16,), jnp.float32)
        lane = lax.iota(jnp.int32, 16)

        def step(off):
            c16 = conf_v[pl.ds(off, 16)]
            a16 = acc_v[pl.ds(off, 16)]
            b = jnp.minimum((c16 * float(N_BINS)).astype(jnp.int32), N_BINS - 1)
            idx = b * 16 + lane
            plsc.addupdate_scatter(hist_v, [idx], ones)
            plsc.addupdate_scatter(hist_v, [idx + 256], c16)
            plsc.addupdate_scatter(hist_v, [idx + 512], a16)

        unroll = 4
        def body(i, carry):
            for u in range(unroll):
                step(i * (16 * unroll) + u * 16)
            return carry

        lax.fori_loop(0, chunk // (16 * unroll), body, 0)
        pltpu.sync_copy(hist_v, out_hbm.at[pl.ds(wid * 768, 768)])

    return hist(conf, acc).reshape(nw, 3, 16, 16)


def kernel(logits, labels):
    n, c = logits.shape
    labels32 = labels.astype(jnp.int32)
    conf, acc = _tc_stats(logits, labels32, block_rows=8192)
    parts = _sc_hist(conf, acc)
    stats = parts.sum(axis=(0, 3))
    cnt = stats[0, :N_BINS]
    sconf = stats[1, :N_BINS]
    sacc = stats[2, :N_BINS]
    safe = jnp.maximum(cnt, 1.0)
    term = jnp.abs(sconf / safe - sacc / safe) * (cnt / n)
    ece = jnp.sum(jnp.where(cnt > 0, term, 0.0))
    return ece.reshape(1)


# speedup vs baseline: 11.9730x; 1.2515x over previous
import functools

import jax
import jax.numpy as jnp
from jax import lax
from jax.experimental import pallas as pl
from jax.experimental.pallas import tpu as pltpu
from jax.experimental.pallas import tpu_sc as plsc

N_BINS = 15


def _tc_body(lt_ref, labels_ref, out_ref):
    xt = lt_ref[...]
    c, r = xt.shape
    m = jnp.max(xt, axis=0, keepdims=True)
    e = jnp.exp(xt - m)
    ones = jnp.ones((1, c), jnp.float32)
    s = lax.dot_general(ones, e, (((1,), (0,)), ((), ())),
                        preferred_element_type=jnp.float32)
    conf = 1.0 / s
    mask = (xt == m).astype(jnp.float32)
    ci = lax.broadcasted_iota(jnp.int32, (1, c), 1)
    w = lax.bitcast_convert_type((127 + 63 - ci) << 23, jnp.float32)
    pv = lax.dot_general(w, mask, (((1,), (0,)), ((), ())),
                         preferred_element_type=jnp.float32)
    ex = (lax.bitcast_convert_type(pv, jnp.int32) >> 23) - 127
    pred = 63 - ex
    lab = labels_ref[0]
    out_ref[0] = jnp.where(pred == lab, -conf, conf)


def _tc_stats(logits, labels32, block_rows):
    n, c = logits.shape
    g = n // block_rows
    lt = logits.T
    labels3 = labels32.reshape(g, 1, block_rows)
    packed = pl.pallas_call(
        _tc_body,
        grid=(g,),
        in_specs=[
            pl.BlockSpec((c, block_rows), lambda i: (0, i)),
            pl.BlockSpec((1, 1, block_rows), lambda i: (i, 0, 0)),
        ],
        out_specs=pl.BlockSpec((1, 1, block_rows), lambda i: (i, 0, 0)),
        out_shape=jax.ShapeDtypeStruct((g, 1, block_rows), jnp.float32),
        compiler_params=pltpu.CompilerParams(
            dimension_semantics=("arbitrary",)),
    )(lt, labels3)
    return packed.reshape(n)


def _sc_hist(packed):
    n = packed.shape[0]
    info = plsc.get_sparse_core_info()
    nc, ns = info.num_cores, info.num_subcores
    nw = nc * ns
    chunk = n // nw

    mesh = plsc.VectorSubcoreMesh(core_axis_name="c", subcore_axis_name="s")

    @functools.partial(
        pl.kernel,
        mesh=mesh,
        out_type=jax.ShapeDtypeStruct((nw * 512,), jnp.float32),
        compiler_params=pltpu.CompilerParams(needs_layout_passes=False),
        scratch_types=[
            pltpu.VMEM((chunk,), jnp.float32),
            pltpu.VMEM((512,), jnp.float32),
        ],
    )
    def hist(packed_hbm, out_hbm, packed_v, hist_v):
        wid = lax.axis_index("s") * nc + lax.axis_index("c")
        base = wid * chunk
        pltpu.sync_copy(packed_hbm.at[pl.ds(base, chunk)], packed_v)
        zeros = jnp.zeros((16,), jnp.float32)
        for j in range(32):
            hist_v[pl.ds(j * 16, 16)] = zeros
        lane = lax.iota(jnp.int32, 16)

        def step(off):
            p16 = packed_v[pl.ds(off, 16)]
            c16 = jnp.abs(p16)
            ca16 = jnp.where(p16 < 0.0, 4097.0, 1.0)
            b = jnp.minimum((c16 * float(N_BINS)).astype(jnp.int32), N_BINS - 1)
            idx = b * 16 + lane
            plsc.addupdate_scatter(hist_v, [idx], ca16)
            plsc.addupdate_scatter(hist_v, [idx + 256], c16)

        unroll = 4
        def body(i, carry):
            for u in range(unroll):
                step(i * (16 * unroll) + u * 16)
            return carry

        lax.fori_loop(0, chunk // (16 * unroll), body, 0)
        pltpu.sync_copy(hist_v, out_hbm.at[pl.ds(wid * 512, 512)])

    return hist(packed).reshape(nw, 2, 16, 16)


def kernel(logits, labels):
    n, c = logits.shape
    labels32 = labels.astype(jnp.int32)
    packed = _tc_stats(logits, labels32, block_rows=16384)
    parts = _sc_hist(packed)
    ca = parts[:, 0]
    sacc_p = jnp.floor(ca * (1.0 / 4096.0))
    cnt_p = ca - 4096.0 * sacc_p
    cnt = cnt_p.sum(axis=(0, 2))[:N_BINS]
    sacc = sacc_p.sum(axis=(0, 2))[:N_BINS]
    sconf = parts[:, 1].sum(axis=(0, 2))[:N_BINS]
    safe = jnp.maximum(cnt, 1.0)
    term = jnp.abs(sconf / safe - sacc / safe) * (cnt / n)
    ece = jnp.sum(jnp.where(cnt > 0, term, 0.0))
    return ece.reshape(1)


# speedup vs baseline: 12.8650x; 1.0745x over previous
import functools

import jax
import jax.numpy as jnp
from jax import lax
from jax.experimental import pallas as pl
from jax.experimental.pallas import tpu as pltpu
from jax.experimental.pallas import tpu_sc as plsc

N_BINS = 15


def _tc_body(lt_ref, labels_ref, out_ref):
    xt = lt_ref[...]
    c, r = xt.shape
    m = jnp.max(xt, axis=0, keepdims=True)
    e = jnp.exp(xt - m)
    ones = jnp.ones((1, c), jnp.float32)
    s = lax.dot_general(ones, e, (((1,), (0,)), ((), ())),
                        preferred_element_type=jnp.float32)
    conf = 1.0 / s
    mask = (xt == m).astype(jnp.float32)
    ci = lax.broadcasted_iota(jnp.int32, (1, c), 1)
    w = lax.bitcast_convert_type((127 + 63 - ci) << 23, jnp.float32)
    pv = lax.dot_general(w, mask, (((1,), (0,)), ((), ())),
                         preferred_element_type=jnp.float32)
    ex = (lax.bitcast_convert_type(pv, jnp.int32) >> 23) - 127
    pred = 63 - ex
    lab = labels_ref[0]
    out_ref[0] = jnp.where(pred == lab, -conf, conf)


def _tc_stats(lt, labels3, block_rows, g_slice, base):
    c = lt.shape[0]
    packed = pl.pallas_call(
        _tc_body,
        grid=(g_slice,),
        in_specs=[
            pl.BlockSpec((c, block_rows), lambda i: (0, base + i)),
            pl.BlockSpec((1, 1, block_rows), lambda i: (base + i, 0, 0)),
        ],
        out_specs=pl.BlockSpec((1, 1, block_rows), lambda i: (i, 0, 0)),
        out_shape=jax.ShapeDtypeStruct((g_slice, 1, block_rows), jnp.float32),
        compiler_params=pltpu.CompilerParams(
            dimension_semantics=("arbitrary",)),
    )(lt, labels3)
    return packed.reshape(g_slice * block_rows)


def _sc_hist(packed):
    n = packed.shape[0]
    info = plsc.get_sparse_core_info()
    nc, ns = info.num_cores, info.num_subcores
    nw = nc * ns
    chunk = n // nw

    mesh = plsc.VectorSubcoreMesh(core_axis_name="c", subcore_axis_name="s")

    @functools.partial(
        pl.kernel,
        mesh=mesh,
        out_type=jax.ShapeDtypeStruct((nw * 512,), jnp.float32),
        compiler_params=pltpu.CompilerParams(needs_layout_passes=False),
        scratch_types=[
            pltpu.VMEM((chunk,), jnp.float32),
            pltpu.VMEM((512,), jnp.float32),
        ],
    )
    def hist(packed_hbm, out_hbm, packed_v, hist_v):
        wid = lax.axis_index("s") * nc + lax.axis_index("c")
        base = wid * chunk
        pltpu.sync_copy(packed_hbm.at[pl.ds(base, chunk)], packed_v)
        zeros = jnp.zeros((16,), jnp.float32)
        for j in range(32):
            hist_v[pl.ds(j * 16, 16)] = zeros
        lane = lax.iota(jnp.int32, 16)

        def step(off):
            p16 = packed_v[pl.ds(off, 16)]
            c16 = jnp.abs(p16)
            ca16 = jnp.where(p16 < 0.0, 4097.0, 1.0)
            b = jnp.minimum((c16 * float(N_BINS)).astype(jnp.int32), N_BINS - 1)
            idx = b * 16 + lane
            plsc.addupdate_scatter(hist_v, [idx], ca16)
            plsc.addupdate_scatter(hist_v, [idx + 256], c16)

        unroll = 4
        def body(i, carry):
            for u in range(unroll):
                step(i * (16 * unroll) + u * 16)
            return carry

        lax.fori_loop(0, chunk // (16 * unroll), body, 0)
        pltpu.sync_copy(hist_v, out_hbm.at[pl.ds(wid * 512, 512)])

    return hist(packed).reshape(nw, 2, 16, 16)


def kernel(logits, labels):
    n, c = logits.shape
    n_slices = 2
    block_rows = 16384
    g = n // block_rows
    g_slice = g // n_slices
    lt = logits.T
    labels3 = labels.astype(jnp.int32).reshape(g, 1, block_rows)
    parts = []
    for si in range(n_slices):
        packed = _tc_stats(lt, labels3, block_rows, g_slice, si * g_slice)
        parts.append(_sc_hist(packed))
    parts = jnp.stack(parts)
    ca = parts[:, :, 0]
    sacc_p = jnp.floor(ca * (1.0 / 4096.0))
    cnt_p = ca - 4096.0 * sacc_p
    cnt = cnt_p.sum(axis=(0, 1, 3))[:N_BINS]
    sacc = sacc_p.sum(axis=(0, 1, 3))[:N_BINS]
    sconf = parts[:, :, 1].sum(axis=(0, 1, 3))[:N_BINS]
    safe = jnp.maximum(cnt, 1.0)
    term = jnp.abs(sconf / safe - sacc / safe) * (cnt / n)
    ece = jnp.sum(jnp.where(cnt > 0, term, 0.0))
    return ece.reshape(1)


# speedup vs baseline: 13.3033x; 1.0341x over previous
import functools

import jax
import jax.numpy as jnp
from jax import lax
from jax.experimental import pallas as pl
from jax.experimental.pallas import tpu as pltpu
from jax.experimental.pallas import tpu_sc as plsc

N_BINS = 15


def _tc_body(lt_ref, labels_ref, out_ref):
    xt = lt_ref[...]
    c, r = xt.shape
    m = jnp.max(xt, axis=0, keepdims=True)
    e = jnp.exp(xt - m)
    ones = jnp.ones((1, c), jnp.float32)
    s = lax.dot_general(ones, e, (((1,), (0,)), ((), ())),
                        preferred_element_type=jnp.float32)
    conf = 1.0 / s
    mask = (xt == m).astype(jnp.float32)
    ci = lax.broadcasted_iota(jnp.int32, (1, c), 1)
    w = lax.bitcast_convert_type((127 + 63 - ci) << 23, jnp.float32)
    pv = lax.dot_general(w, mask, (((1,), (0,)), ((), ())),
                         preferred_element_type=jnp.float32)
    ex = (lax.bitcast_convert_type(pv, jnp.int32) >> 23) - 127
    pred = 63 - ex
    lab = labels_ref[0]
    out_ref[0] = jnp.where(pred == lab, -conf, conf)


def _tc_stats(lt, labels3, block_rows, g_slice, base):
    c = lt.shape[0]
    packed = pl.pallas_call(
        _tc_body,
        grid=(g_slice,),
        in_specs=[
            pl.BlockSpec((c, block_rows), lambda i: (0, base + i)),
            pl.BlockSpec((1, 1, block_rows), lambda i: (base + i, 0, 0)),
        ],
        out_specs=pl.BlockSpec((1, 1, block_rows), lambda i: (i, 0, 0)),
        out_shape=jax.ShapeDtypeStruct((g_slice, 1, block_rows), jnp.float32),
        compiler_params=pltpu.CompilerParams(
            dimension_semantics=("arbitrary",)),
    )(lt, labels3)
    return packed.reshape(g_slice * block_rows)


def _sc_hist(packed):
    n = packed.shape[0]
    info = plsc.get_sparse_core_info()
    nc, ns = info.num_cores, info.num_subcores
    nw = nc * ns
    chunk = n // nw

    mesh = plsc.VectorSubcoreMesh(core_axis_name="c", subcore_axis_name="s")

    @functools.partial(
        pl.kernel,
        mesh=mesh,
        out_type=jax.ShapeDtypeStruct((nw * 512,), jnp.float32),
        compiler_params=pltpu.CompilerParams(needs_layout_passes=False),
        scratch_types=[
            pltpu.VMEM((chunk,), jnp.float32),
            pltpu.VMEM((512,), jnp.float32),
        ],
    )
    def hist(packed_hbm, out_hbm, packed_v, hist_v):
        wid = lax.axis_index("s") * nc + lax.axis_index("c")
        base = wid * chunk
        pltpu.sync_copy(packed_hbm.at[pl.ds(base, chunk)], packed_v)
        zeros = jnp.zeros((16,), jnp.float32)
        for j in range(32):
            hist_v[pl.ds(j * 16, 16)] = zeros
        lane = lax.iota(jnp.int32, 16)

        def step(off):
            p16 = packed_v[pl.ds(off, 16)]
            c16 = jnp.abs(p16)
            ca16 = jnp.where(p16 < 0.0, 4097.0, 1.0)
            b = jnp.minimum((c16 * float(N_BINS)).astype(jnp.int32), N_BINS - 1)
            idx = b * 16 + lane
            plsc.addupdate_scatter(hist_v, [idx], ca16)
            plsc.addupdate_scatter(hist_v, [idx + 256], c16)

        unroll = 4
        def body(i, carry):
            for u in range(unroll):
                step(i * (16 * unroll) + u * 16)
            return carry

        lax.fori_loop(0, chunk // (16 * unroll), body, 0)
        pltpu.sync_copy(hist_v, out_hbm.at[pl.ds(wid * 512, 512)])

    return hist(packed).reshape(nw, 2, 16, 16)


def kernel(logits, labels):
    n, c = logits.shape
    n_slices = 4
    block_rows = 16384
    g = n // block_rows
    g_slice = g // n_slices
    lt = logits.T
    labels3 = labels.astype(jnp.int32).reshape(g, 1, block_rows)
    parts = []
    for si in range(n_slices):
        packed = _tc_stats(lt, labels3, block_rows, g_slice, si * g_slice)
        parts.append(_sc_hist(packed))
    parts = jnp.stack(parts)
    ca = parts[:, :, 0]
    sacc_p = jnp.floor(ca * (1.0 / 4096.0))
    cnt_p = ca - 4096.0 * sacc_p
    cnt = cnt_p.sum(axis=(0, 1, 3))[:N_BINS]
    sacc = sacc_p.sum(axis=(0, 1, 3))[:N_BINS]
    sconf = parts[:, :, 1].sum(axis=(0, 1, 3))[:N_BINS]
    safe = jnp.maximum(cnt, 1.0)
    term = jnp.abs(sconf / safe - sacc / safe) * (cnt / n)
    ece = jnp.sum(jnp.where(cnt > 0, term, 0.0))
    return ece.reshape(1)


# speedup vs baseline: 14.2865x; 1.0739x over previous
import functools

import jax
import jax.numpy as jnp
from jax import lax
from jax.experimental import pallas as pl
from jax.experimental.pallas import tpu as pltpu
from jax.experimental.pallas import tpu_sc as plsc

N_BINS = 15


def _tc_body(lt_ref, labels_ref, out_ref):
    xt = lt_ref[...]
    c, r = xt.shape
    m = jnp.max(xt, axis=0, keepdims=True)
    e = jnp.exp(xt - m)
    ones = jnp.ones((1, c), jnp.float32)
    s = lax.dot_general(ones, e, (((1,), (0,)), ((), ())),
                        preferred_element_type=jnp.float32)
    conf = 1.0 / s
    mask = (xt == m).astype(jnp.float32)
    ci = lax.broadcasted_iota(jnp.int32, (1, c), 1)
    w = lax.bitcast_convert_type((127 + 63 - ci) << 23, jnp.float32)
    pv = lax.dot_general(w, mask, (((1,), (0,)), ((), ())),
                         preferred_element_type=jnp.float32)
    ex = (lax.bitcast_convert_type(pv, jnp.int32) >> 23) - 127
    pred = 63 - ex
    lab = labels_ref[0]
    out_ref[0] = jnp.where(pred == lab, -conf, conf)


def _tc_stats(lt, labels3, block_rows, g_slice, base):
    c = lt.shape[0]
    packed = pl.pallas_call(
        _tc_body,
        grid=(g_slice,),
        in_specs=[
            pl.BlockSpec((c, block_rows), lambda i: (0, base + i)),
            pl.BlockSpec((1, 1, block_rows), lambda i: (base + i, 0, 0)),
        ],
        out_specs=pl.BlockSpec((1, 1, block_rows), lambda i: (i, 0, 0)),
        out_shape=jax.ShapeDtypeStruct((g_slice, 1, block_rows), jnp.float32),
        compiler_params=pltpu.CompilerParams(
            dimension_semantics=("arbitrary",)),
    )(lt, labels3)
    return packed.reshape(g_slice * block_rows)


def _sc_hist(packed):
    n = packed.shape[0]
    info = plsc.get_sparse_core_info()
    nc, ns = info.num_cores, info.num_subcores
    nw = nc * ns
    chunk = n // nw

    mesh = plsc.VectorSubcoreMesh(core_axis_name="c", subcore_axis_name="s")

    @functools.partial(
        pl.kernel,
        mesh=mesh,
        out_type=jax.ShapeDtypeStruct((nw * 512,), jnp.float32),
        compiler_params=pltpu.CompilerParams(needs_layout_passes=False),
        scratch_types=[
            pltpu.VMEM((chunk,), jnp.float32),
            pltpu.VMEM((512,), jnp.float32),
        ],
    )
    def hist(packed_hbm, out_hbm, packed_v, hist_v):
        wid = lax.axis_index("s") * nc + lax.axis_index("c")
        base = wid * chunk
        pltpu.sync_copy(packed_hbm.at[pl.ds(base, chunk)], packed_v)
        zeros = jnp.zeros((16,), jnp.float32)
        for j in range(32):
            hist_v[pl.ds(j * 16, 16)] = zeros
        lane = lax.iota(jnp.int32, 16)

        def step(off):
            p16 = packed_v[pl.ds(off, 16)]
            c16 = jnp.abs(p16)
            ca16 = jnp.where(p16 < 0.0, 4097.0, 1.0)
            b = jnp.minimum((c16 * float(N_BINS)).astype(jnp.int32), N_BINS - 1)
            idx = b * 16 + lane
            plsc.addupdate_scatter(hist_v, [idx], ca16)
            plsc.addupdate_scatter(hist_v, [idx + 256], c16)

        unroll = 4
        def body(i, carry):
            for u in range(unroll):
                step(i * (16 * unroll) + u * 16)
            return carry

        lax.fori_loop(0, chunk // (16 * unroll), body, 0)
        pltpu.sync_copy(hist_v, out_hbm.at[pl.ds(wid * 512, 512)])

    return hist(packed).reshape(nw, 2, 16, 16)


def kernel(logits, labels):
    n, c = logits.shape
    n_slices = 4
    block_rows = 32768
    g = n // block_rows
    g_slice = g // n_slices
    lt = logits.T
    labels3 = labels.astype(jnp.int32).reshape(g, 1, block_rows)
    parts = []
    for si in range(n_slices):
        packed = _tc_stats(lt, labels3, block_rows, g_slice, si * g_slice)
        parts.append(_sc_hist(packed))
    parts = jnp.stack(parts)
    ca = parts[:, :, 0]
    sacc_p = jnp.floor(ca * (1.0 / 4096.0))
    cnt_p = ca - 4096.0 * sacc_p
    cnt = cnt_p.sum(axis=(0, 1, 3))[:N_BINS]
    sacc = sacc_p.sum(axis=(0, 1, 3))[:N_BINS]
    sconf = parts[:, :, 1].sum(axis=(0, 1, 3))[:N_BINS]
    safe = jnp.maximum(cnt, 1.0)
    term = jnp.abs(sconf / safe - sacc / safe) * (cnt / n)
    ece = jnp.sum(jnp.where(cnt > 0, term, 0.0))
    return ece.reshape(1)
